# baseline (device time: 287378 ns/iter reference)
import jax
import jax.numpy as jnp
from jax import lax
from jax.experimental import pallas as pl
from jax.experimental.pallas import tpu as pltpu

M = 8192
D = 2048
HALF = M // 2
MROWS = 256
N_MSGS = HALF // MROWS
SLOTS = 3
OS = 2
US = 2
AG_LAG = 1


def kernel(partial, resid, gamma):
    def body(partial_ref, resid_ref, gamma_ref, out_ref,
             recv_buf, ag_recv_buf, a_bufs, r_bufs,
             rs_send_bufs, o_bufs, o_send_bufs, u_bufs,
             rs_send_sems, rs_recv_sems, ag_send_sems, ag_recv_sems,
             store_sems, ustore_sems, local_sems):
        my_x = lax.axis_index("x")
        my_y = lax.axis_index("y")
        y_nbr = (my_x, 1 - my_y)
        x_nbr = (1 - my_x, my_y)
        base = my_x * HALF
        nbr_base = (1 - my_x) * HALF

        barrier = pltpu.get_barrier_semaphore()
        for nbr in (y_nbr, x_nbr):
            pl.semaphore_signal(barrier, inc=1, device_id=nbr,
                                device_id_type=pl.DeviceIdType.MESH)
        pl.semaphore_wait(barrier, 2)

        cp_a = pltpu.make_async_copy(
            partial_ref.at[0, pl.ds(base, MROWS), :], a_bufs.at[0],
            local_sems.at[0])
        cp_r = pltpu.make_async_copy(
            resid_ref.at[pl.ds(base, MROWS), :], r_bufs.at[0],
            local_sems.at[1])
        cp_a.start()
        cp_r.start()
        local_descs = {0: (cp_a, cp_r)}

        rs_descs, ag_descs, st_descs, ust_descs = {}, {}, {}, {}

        def process_ag_recv(d):
            recv = pltpu.make_async_remote_copy(
                src_ref=o_send_bufs.at[0],
                dst_ref=ag_recv_buf.at[d],
                send_sem=ag_send_sems.at[0],
                recv_sem=ag_recv_sems.at[d],
                device_id=x_nbr,
                device_id_type=pl.DeviceIdType.MESH,
            )
            recv.wait_recv()
            su = d % US
            if d >= US:
                ust_descs.pop(d - US).wait()
            u_bufs[su] = ag_recv_buf[d].astype(jnp.float32)
            ust = pltpu.make_async_copy(
                u_bufs.at[su],
                out_ref.at[pl.ds(nbr_base + d * MROWS, MROWS), :],
                ustore_sems.at[su])
            ust.start()
            ust_descs[d] = ust

        for m in range(N_MSGS):
            s = m % 2
            ms_slot = m % SLOTS
            so = m % OS
            row0 = base + m * MROWS
            la, lr = local_descs.pop(m)
            la.wait()
            lr.wait()
            if m + 1 < N_MSGS:
                nrow0 = base + (m + 1) * MROWS
                na = pltpu.make_async_copy(
                    partial_ref.at[0, pl.ds(nrow0, MROWS), :],
                    a_bufs.at[1 - s], local_sems.at[2 * (1 - s)])
                nr = pltpu.make_async_copy(
                    resid_ref.at[pl.ds(nrow0, MROWS), :],
                    r_bufs.at[1 - s], local_sems.at[2 * (1 - s) + 1])
                na.start()
                nr.start()
                local_descs[m + 1] = (na, nr)
            if m >= SLOTS:
                rs_descs.pop(m - SLOTS).wait_send()
            rs_send_bufs[ms_slot] = a_bufs[s].astype(jnp.bfloat16)
            rs = pltpu.make_async_remote_copy(
                src_ref=rs_send_bufs.at[ms_slot],
                dst_ref=recv_buf.at[m],
                send_sem=rs_send_sems.at[ms_slot],
                recv_sem=rs_recv_sems.at[m],
                device_id=y_nbr,
                device_id_type=pl.DeviceIdType.MESH,
            )
            rs.start()
            rs_descs[m] = rs
            rs.wait_recv()
            if m >= OS:
                st_descs.pop(m - OS).wait()
            if m >= SLOTS:
                ag_descs.pop(m - SLOTS).wait_send()
            yv = a_bufs[s] + recv_buf[m].astype(jnp.float32) + r_bufs[s]
            msq = jnp.mean(yv * yv, axis=-1, keepdims=True)
            res = yv * lax.rsqrt(msq + 1e-6) * gamma_ref[...]
            o_bufs[so] = res
            o_send_bufs[ms_slot] = res.astype(jnp.bfloat16)
            ag = pltpu.make_async_remote_copy(
                src_ref=o_send_bufs.at[ms_slot],
                dst_ref=ag_recv_buf.at[m],
                send_sem=ag_send_sems.at[ms_slot],
                recv_sem=ag_recv_sems.at[m],
                device_id=x_nbr,
                device_id_type=pl.DeviceIdType.MESH,
            )
            ag.start()
            ag_descs[m] = ag
            st = pltpu.make_async_copy(
                o_bufs.at[so], out_ref.at[pl.ds(row0, MROWS), :],
                store_sems.at[so])
            st.start()
            st_descs[m] = st
            if m >= AG_LAG:
                process_ag_recv(m - AG_LAG)

        for d in range(N_MSGS - AG_LAG, N_MSGS):
            process_ag_recv(d)
        for k in sorted(rs_descs):
            rs_descs[k].wait_send()
        for k in sorted(ag_descs):
            ag_descs[k].wait_send()
        for k in sorted(st_descs):
            st_descs[k].wait()
        for k in sorted(ust_descs):
            ust_descs[k].wait()

    return pl.pallas_call(
        body,
        out_shape=jax.ShapeDtypeStruct((M, D), jnp.float32),
        in_specs=[
            pl.BlockSpec(memory_space=pl.ANY),
            pl.BlockSpec(memory_space=pl.ANY),
            pl.BlockSpec(memory_space=pltpu.VMEM),
        ],
        out_specs=pl.BlockSpec(memory_space=pl.ANY),
        scratch_shapes=[
            pltpu.VMEM((N_MSGS, MROWS, D), jnp.bfloat16),
            pltpu.VMEM((N_MSGS, MROWS, D), jnp.bfloat16),
            pltpu.VMEM((2, MROWS, D), jnp.float32),
            pltpu.VMEM((2, MROWS, D), jnp.float32),
            pltpu.VMEM((SLOTS, MROWS, D), jnp.bfloat16),
            pltpu.VMEM((OS, MROWS, D), jnp.float32),
            pltpu.VMEM((SLOTS, MROWS, D), jnp.bfloat16),
            pltpu.VMEM((US, MROWS, D), jnp.float32),
            pltpu.SemaphoreType.DMA((SLOTS,)),
            pltpu.SemaphoreType.DMA((N_MSGS,)),
            pltpu.SemaphoreType.DMA((SLOTS,)),
            pltpu.SemaphoreType.DMA((N_MSGS,)),
            pltpu.SemaphoreType.DMA((OS,)),
            pltpu.SemaphoreType.DMA((US,)),
            pltpu.SemaphoreType.DMA((4,)),
        ],
        compiler_params=pltpu.CompilerParams(
            collective_id=0,
            vmem_limit_bytes=60 * 1024 * 1024,
        ),
    )(partial, resid, gamma)


# device time: 246667 ns/iter; 1.1650x vs baseline; 1.1650x over previous
import jax
import jax.numpy as jnp
from jax import lax
from jax.experimental import pallas as pl
from jax.experimental.pallas import tpu as pltpu

M = 8192
D = 2048
HALF = M // 2
MROWS = 256
N_MSGS = HALF // MROWS
SLOTS = 3
OS = 2
US = 2
AG_LAG = 1


def kernel(partial, resid, gamma):
    def body(partial_ref, resid_ref, gamma_ref, out_ref,
             recv_buf, ag_recv_buf, a_bufs, r_bufs,
             rs_send_bufs, o_bufs, o_send_bufs, u_bufs,
             rs_send_sems, rs_recv_sems, ag_send_sems, ag_recv_sems,
             store_sems, ustore_sems, local_sems):
        my_x = lax.axis_index("x")
        my_y = lax.axis_index("y")
        y_nbr = (my_x, 1 - my_y)
        x_nbr = (1 - my_x, my_y)
        base = my_x * HALF
        nbr_base = (1 - my_x) * HALF

        barrier = pltpu.get_barrier_semaphore()
        for nbr in (y_nbr, x_nbr):
            pl.semaphore_signal(barrier, inc=1, device_id=nbr,
                                device_id_type=pl.DeviceIdType.MESH)
        pl.semaphore_wait(barrier, 2)

        local_descs = {}
        for i in range(min(2, N_MSGS)):
            cp_a = pltpu.make_async_copy(
                partial_ref.at[0, pl.ds(base + i * MROWS, MROWS), :],
                a_bufs.at[i], local_sems.at[2 * i])
            cp_r = pltpu.make_async_copy(
                resid_ref.at[pl.ds(base + i * MROWS, MROWS), :],
                r_bufs.at[i], local_sems.at[2 * i + 1])
            cp_a.start()
            cp_r.start()
            local_descs[i] = (cp_a, cp_r)

        rs_descs, ag_descs, st_descs, ust_descs = {}, {}, {}, {}

        def send_rs(k):
            sk = k % 2
            la, lr = local_descs.pop(k)
            la.wait()
            lr.wait()
            if k >= SLOTS:
                rs_descs.pop(k - SLOTS).wait_send()
            rs_send_bufs[k % SLOTS] = a_bufs[sk].astype(jnp.bfloat16)
            rs = pltpu.make_async_remote_copy(
                src_ref=rs_send_bufs.at[k % SLOTS],
                dst_ref=recv_buf.at[k],
                send_sem=rs_send_sems.at[k % SLOTS],
                recv_sem=rs_recv_sems.at[k],
                device_id=y_nbr,
                device_id_type=pl.DeviceIdType.MESH,
            )
            rs.start()
            rs_descs[k] = rs

        send_rs(0)

        def process_ag_recv(d):
            recv = pltpu.make_async_remote_copy(
                src_ref=o_send_bufs.at[0],
                dst_ref=ag_recv_buf.at[d],
                send_sem=ag_send_sems.at[0],
                recv_sem=ag_recv_sems.at[d],
                device_id=x_nbr,
                device_id_type=pl.DeviceIdType.MESH,
            )
            recv.wait_recv()
            su = d % US
            if d >= US:
                ust_descs.pop(d - US).wait()
            u_bufs[su] = ag_recv_buf[d].astype(jnp.float32)
            ust = pltpu.make_async_copy(
                u_bufs.at[su],
                out_ref.at[pl.ds(nbr_base + d * MROWS, MROWS), :],
                ustore_sems.at[su])
            ust.start()
            ust_descs[d] = ust

        for m in range(N_MSGS):
            s = m % 2
            ms_slot = m % SLOTS
            so = m % OS
            row0 = base + m * MROWS
            if m + 1 < N_MSGS:
                send_rs(m + 1)
            rs_descs[m].wait_recv()
            if m >= OS:
                st_descs.pop(m - OS).wait()
            if m >= SLOTS:
                ag_descs.pop(m - SLOTS).wait_send()
            yv = a_bufs[s] + recv_buf[m].astype(jnp.float32) + r_bufs[s]
            msq = jnp.mean(yv * yv, axis=-1, keepdims=True)
            res = yv * lax.rsqrt(msq + 1e-6) * gamma_ref[...]
            o_bufs[so] = res
            o_send_bufs[ms_slot] = res.astype(jnp.bfloat16)
            ag = pltpu.make_async_remote_copy(
                src_ref=o_send_bufs.at[ms_slot],
                dst_ref=ag_recv_buf.at[m],
                send_sem=ag_send_sems.at[ms_slot],
                recv_sem=ag_recv_sems.at[m],
                device_id=x_nbr,
                device_id_type=pl.DeviceIdType.MESH,
            )
            ag.start()
            ag_descs[m] = ag
            st = pltpu.make_async_copy(
                o_bufs.at[so], out_ref.at[pl.ds(row0, MROWS), :],
                store_sems.at[so])
            st.start()
            st_descs[m] = st
            if m + 2 < N_MSGS:
                nrow0 = base + (m + 2) * MROWS
                na = pltpu.make_async_copy(
                    partial_ref.at[0, pl.ds(nrow0, MROWS), :],
                    a_bufs.at[s], local_sems.at[2 * s])
                nr = pltpu.make_async_copy(
                    resid_ref.at[pl.ds(nrow0, MROWS), :],
                    r_bufs.at[s], local_sems.at[2 * s + 1])
                na.start()
                nr.start()
                local_descs[m + 2] = (na, nr)
            if m >= AG_LAG:
                process_ag_recv(m - AG_LAG)

        for d in range(N_MSGS - AG_LAG, N_MSGS):
            process_ag_recv(d)
        for k in sorted(rs_descs):
            rs_descs[k].wait_send()
        for k in sorted(ag_descs):
            ag_descs[k].wait_send()
        for k in sorted(st_descs):
            st_descs[k].wait()
        for k in sorted(ust_descs):
            ust_descs[k].wait()

    return pl.pallas_call(
        body,
        out_shape=jax.ShapeDtypeStruct((M, D), jnp.float32),
        in_specs=[
            pl.BlockSpec(memory_space=pl.ANY),
            pl.BlockSpec(memory_space=pl.ANY),
            pl.BlockSpec(memory_space=pltpu.VMEM),
        ],
        out_specs=pl.BlockSpec(memory_space=pl.ANY),
        scratch_shapes=[
            pltpu.VMEM((N_MSGS, MROWS, D), jnp.bfloat16),
            pltpu.VMEM((N_MSGS, MROWS, D), jnp.bfloat16),
            pltpu.VMEM((2, MROWS, D), jnp.float32),
            pltpu.VMEM((2, MROWS, D), jnp.float32),
            pltpu.VMEM((SLOTS, MROWS, D), jnp.bfloat16),
            pltpu.VMEM((OS, MROWS, D), jnp.float32),
            pltpu.VMEM((SLOTS, MROWS, D), jnp.bfloat16),
            pltpu.VMEM((US, MROWS, D), jnp.float32),
            pltpu.SemaphoreType.DMA((SLOTS,)),
            pltpu.SemaphoreType.DMA((N_MSGS,)),
            pltpu.SemaphoreType.DMA((SLOTS,)),
            pltpu.SemaphoreType.DMA((N_MSGS,)),
            pltpu.SemaphoreType.DMA((OS,)),
            pltpu.SemaphoreType.DMA((US,)),
            pltpu.SemaphoreType.DMA((4,)),
        ],
        compiler_params=pltpu.CompilerParams(
            collective_id=0,
            vmem_limit_bytes=60 * 1024 * 1024,
        ),
    )(partial, resid, gamma)


# device time: 246117 ns/iter; 1.1676x vs baseline; 1.0022x over previous
import jax
import jax.numpy as jnp
from jax import lax
from jax.experimental import pallas as pl
from jax.experimental.pallas import tpu as pltpu

M = 8192
D = 2048
HALF = M // 2
MROWS = 256
N_MSGS = HALF // MROWS
SLOTS = 3
OS = 2
US = 2
AG_LAG = 1


def kernel(partial, resid, gamma):
    def body(partial_ref, resid_ref, gamma_ref, out_ref,
             recv_buf, ag_recv_buf, a_bufs, r_bufs,
             rs_send_bufs, o_bufs, o_send_bufs, u_bufs,
             rs_send_sems, rs_recv_sems, ag_send_sems, ag_recv_sems,
             store_sems, ustore_sems, la_sems, lr_sems):
        my_x = lax.axis_index("x")
        my_y = lax.axis_index("y")
        y_nbr = (my_x, 1 - my_y)
        x_nbr = (1 - my_x, my_y)
        base = my_x * HALF
        nbr_base = (1 - my_x) * HALF

        a_descs, r_descs = {}, {}

        def load_a(k):
            cp = pltpu.make_async_copy(
                partial_ref.at[0, pl.ds(base + k * MROWS, MROWS), :],
                a_bufs.at[k % 4], la_sems.at[k % 4])
            cp.start()
            a_descs[k] = cp

        def load_r(k):
            cp = pltpu.make_async_copy(
                resid_ref.at[pl.ds(base + k * MROWS, MROWS), :],
                r_bufs.at[k % 2], lr_sems.at[k % 2])
            cp.start()
            r_descs[k] = cp

        for i in range(min(3, N_MSGS)):
            load_a(i)
        for i in range(min(2, N_MSGS)):
            load_r(i)

        rs_descs, ag_descs, st_descs, ust_descs = {}, {}, {}, {}

        def cast_rs(k):
            a_descs.pop(k).wait()
            if k >= SLOTS:
                rs_descs.pop(k - SLOTS).wait_send()
            rs_send_bufs[k % SLOTS] = a_bufs[k % 4].astype(jnp.bfloat16)

        def send_rs(k):
            rs = pltpu.make_async_remote_copy(
                src_ref=rs_send_bufs.at[k % SLOTS],
                dst_ref=recv_buf.at[k],
                send_sem=rs_send_sems.at[k % SLOTS],
                recv_sem=rs_recv_sems.at[k],
                device_id=y_nbr,
                device_id_type=pl.DeviceIdType.MESH,
            )
            rs.start()
            rs_descs[k] = rs

        cast_rs(0)

        barrier = pltpu.get_barrier_semaphore()
        for nbr in (y_nbr, x_nbr):
            pl.semaphore_signal(barrier, inc=1, device_id=nbr,
                                device_id_type=pl.DeviceIdType.MESH)
        pl.semaphore_wait(barrier, 2)

        send_rs(0)
        if N_MSGS > 1:
            cast_rs(1)
            send_rs(1)

        def process_ag_recv(d):
            recv = pltpu.make_async_remote_copy(
                src_ref=o_send_bufs.at[0],
                dst_ref=ag_recv_buf.at[d],
                send_sem=ag_send_sems.at[0],
                recv_sem=ag_recv_sems.at[d],
                device_id=x_nbr,
                device_id_type=pl.DeviceIdType.MESH,
            )
            recv.wait_recv()
            su = d % US
            if d >= US:
                ust_descs.pop(d - US).wait()
            u_bufs[su] = ag_recv_buf[d].astype(jnp.float32)
            ust = pltpu.make_async_copy(
                u_bufs.at[su],
                out_ref.at[pl.ds(nbr_base + d * MROWS, MROWS), :],
                ustore_sems.at[su])
            ust.start()
            ust_descs[d] = ust

        for m in range(N_MSGS):
            s = m % 2
            ms_slot = m % SLOTS
            so = m % OS
            row0 = base + m * MROWS
            if m + 2 < N_MSGS:
                cast_rs(m + 2)
                send_rs(m + 2)
            if m + 3 < N_MSGS:
                load_a(m + 3)
            rs_descs[m].wait_recv()
            r_descs.pop(m).wait()
            if m >= OS:
                st_descs.pop(m - OS).wait()
            if m >= SLOTS:
                ag_descs.pop(m - SLOTS).wait_send()
            yv = (a_bufs[m % 4] + recv_buf[m].astype(jnp.float32)
                  + r_bufs[s])
            msq = jnp.mean(yv * yv, axis=-1, keepdims=True)
            res = yv * lax.rsqrt(msq + 1e-6) * gamma_ref[...]
            o_bufs[so] = res
            o_send_bufs[ms_slot] = res.astype(jnp.bfloat16)
            ag = pltpu.make_async_remote_copy(
                src_ref=o_send_bufs.at[ms_slot],
                dst_ref=ag_recv_buf.at[m],
                send_sem=ag_send_sems.at[ms_slot],
                recv_sem=ag_recv_sems.at[m],
                device_id=x_nbr,
                device_id_type=pl.DeviceIdType.MESH,
            )
            ag.start()
            ag_descs[m] = ag
            st = pltpu.make_async_copy(
                o_bufs.at[so], out_ref.at[pl.ds(row0, MROWS), :],
                store_sems.at[so])
            st.start()
            st_descs[m] = st
            if m + 2 < N_MSGS:
                load_r(m + 2)
            if m >= AG_LAG:
                process_ag_recv(m - AG_LAG)

        for d in range(N_MSGS - AG_LAG, N_MSGS):
            process_ag_recv(d)
        for k in sorted(rs_descs):
            rs_descs[k].wait_send()
        for k in sorted(ag_descs):
            ag_descs[k].wait_send()
        for k in sorted(st_descs):
            st_descs[k].wait()
        for k in sorted(ust_descs):
            ust_descs[k].wait()

    return pl.pallas_call(
        body,
        out_shape=jax.ShapeDtypeStruct((M, D), jnp.float32),
        in_specs=[
            pl.BlockSpec(memory_space=pl.ANY),
            pl.BlockSpec(memory_space=pl.ANY),
            pl.BlockSpec(memory_space=pltpu.VMEM),
        ],
        out_specs=pl.BlockSpec(memory_space=pl.ANY),
        scratch_shapes=[
            pltpu.VMEM((N_MSGS, MROWS, D), jnp.bfloat16),
            pltpu.VMEM((N_MSGS, MROWS, D), jnp.bfloat16),
            pltpu.VMEM((4, MROWS, D), jnp.float32),
            pltpu.VMEM((2, MROWS, D), jnp.float32),
            pltpu.VMEM((SLOTS, MROWS, D), jnp.bfloat16),
            pltpu.VMEM((OS, MROWS, D), jnp.float32),
            pltpu.VMEM((SLOTS, MROWS, D), jnp.bfloat16),
            pltpu.VMEM((US, MROWS, D), jnp.float32),
            pltpu.SemaphoreType.DMA((SLOTS,)),
            pltpu.SemaphoreType.DMA((N_MSGS,)),
            pltpu.SemaphoreType.DMA((SLOTS,)),
            pltpu.SemaphoreType.DMA((N_MSGS,)),
            pltpu.SemaphoreType.DMA((OS,)),
            pltpu.SemaphoreType.DMA((US,)),
            pltpu.SemaphoreType.DMA((4,)),
            pltpu.SemaphoreType.DMA((2,)),
        ],
        compiler_params=pltpu.CompilerParams(
            collective_id=0,
            vmem_limit_bytes=62 * 1024 * 1024,
        ),
    )(partial, resid, gamma)
